# Initial kernel scaffold; baseline (speedup 1.0000x reference)
import jax
import jax.numpy as jnp
from jax.experimental import pallas as pl


def kernel(x, token_table, beat_table, bar_table):
    raise NotImplementedError("write your pallas kernel here")


# speedup vs baseline: 2.4410x; 2.4410x over previous
import functools

import jax
import jax.numpy as jnp
import numpy as np
from jax import lax
from jax.experimental import pallas as pl
from jax.experimental.pallas import tpu as pltpu
from jax.experimental.pallas import tpu_sc as plsc

VOCAB = 100000
DIM = 64
B = 4096
L = 200

_info = plsc.get_sparse_core_info()
NC, NS = _info.num_cores, _info.num_subcores
NW = NC * NS
SEQ_W = B // NW
ROWS_W = SEQ_W * L
IDX_CHUNK = 100
ROWS_BUF = 400
N_CHUNKS = ROWS_W // ROWS_BUF


def _pos_encoding_np():
    positions = np.arange(L)[:, np.newaxis].astype(np.float64)
    dims = np.arange(DIM)[np.newaxis, :].astype(np.float64)
    angles = positions / np.power(10000.0, 2 * (dims // 2) / DIM)
    angles[:, 0::2] = np.sin(angles[:, 0::2])
    angles[:, 1::2] = np.cos(angles[:, 1::2])
    return angles.astype(np.float32)


_POS_NP = _pos_encoding_np()


def _bias_body(pos_ref, beat_ref, bar_ref, out_ref):
    beat = jnp.tile(beat_ref[...], (L // 4, 1))
    bar = jnp.tile(bar_ref[...], (L // 16 + 1, 1))[:L]
    out_ref[...] = pos_ref[...] + beat + bar


def _sc_body(x_hbm, tok_hbm, bias_hbm, out_hbm, idx_v, buf_v, sem):
    cid = lax.axis_index("c")
    sid = lax.axis_index("s")
    wid = sid * NC + cid

    pltpu.sync_copy(x_hbm.at[wid], idx_v)

    def chunk(j, _):
        pltpu.sync_copy(bias_hbm, buf_v.at[pl.ds(0, L)])
        pltpu.sync_copy(bias_hbm, buf_v.at[pl.ds(L, L)])
        for k in range(ROWS_BUF // IDX_CHUNK):
            pltpu.async_copy(
                tok_hbm.at[idx_v.at[j * (ROWS_BUF // IDX_CHUNK) + k]],
                buf_v.at[pl.ds(k * IDX_CHUNK, IDX_CHUNK)],
                sem, add=True).wait()
        pltpu.sync_copy(buf_v, out_hbm.at[pl.ds(wid * ROWS_W + j * ROWS_BUF,
                                                ROWS_BUF)])
        return _

    lax.fori_loop(0, N_CHUNKS, chunk, 0, unroll=False)


@jax.jit
def _run(x_r, token_table, beat_table, bar_table):
    pos = jnp.asarray(_POS_NP)
    bias = pl.pallas_call(
        _bias_body,
        out_shape=jax.ShapeDtypeStruct((L, DIM), jnp.float32),
    )(pos, beat_table, bar_table)

    mesh = plsc.VectorSubcoreMesh(core_axis_name="c", subcore_axis_name="s")
    f = pl.kernel(
        _sc_body,
        out_type=jax.ShapeDtypeStruct((B * L, DIM), jnp.float32),
        mesh=mesh,
        scratch_types=[
            pltpu.VMEM((ROWS_W // IDX_CHUNK, IDX_CHUNK), jnp.int32),
            pltpu.VMEM((ROWS_BUF, DIM), jnp.float32),
            pltpu.SemaphoreType.DMA,
        ],
        compiler_params=pltpu.CompilerParams(use_tc_tiling_on_sc=False),
        name="rhythm_embed_sc",
    )
    return f(x_r, token_table, bias)


def kernel(x, token_table, beat_table, bar_table):
    x_r = x.astype(jnp.int32).reshape(NW, ROWS_W // IDX_CHUNK, IDX_CHUNK)
    out = _run(x_r, token_table, beat_table, bar_table)
    return out.reshape(B, L, DIM)
